# baseline (speedup 1.0000x reference)
import functools

import jax
import jax.numpy as jnp
from jax import lax
from jax.experimental import pallas as pl
from jax.experimental.pallas import tpu as pltpu
from jax.experimental.pallas import tpu_sc as plsc

_N = 100000
_E = 1600000
_D = 32
_NC = 2
_NS = 16
_HALF = _N // _NC
_ACC_ROWS = 50176
_ZROWS = 56
_B = 80
_K = 5
_EPT = _E // _NS
_NGRP = _EPT // (_B * _K)
_NW = _NC * _NS
_CCH = 2000
_F = 800
_FF = 1600
_BCAP = 1616
_REG = 53600
_EPW = _E // (_NC * _NS)
_EP = _NC * _NS * _REG
_CW = 400
_NCH = _HALF // _CW
_CNT_W = 16

_sc_mesh = plsc.VectorSubcoreMesh(core_axis_name="c", subcore_axis_name="s")
_sc_params = pltpu.CompilerParams(use_tc_tiling_on_sc=False)
_sc_params_nl = pltpu.CompilerParams(use_tc_tiling_on_sc=False,
                                     needs_layout_passes=False)


def _zero_acc(zbuf, acc, s, width):
    def zrow(k, _):
        for j in range(width // 16):
            zbuf[k, pl.ds(16 * j, 16)] = jnp.zeros((16,), jnp.float32)
        return 0
    lax.fori_loop(0, _ZROWS, zrow, 0)
    rows_pt = _ACC_ROWS // _NS

    def zcpy(k, _):
        pltpu.sync_copy(zbuf, acc.at[pl.ds(s * rows_pt + k * _ZROWS, _ZROWS)])
        return 0
    lax.fori_loop(0, rows_pt // _ZROWS, zcpy, 0)


def _local_dst(dstbuf, dstloc, j, base):
    def chunk(m, _):
        d = dstbuf[pl.ds(j * _B + 16 * m, 16)]
        local = d - base
        ok = (local >= 0) & (local < _HALF)
        dstloc[j, pl.ds(16 * m, 16)] = jnp.where(ok, local, _HALF)
        return 0
    lax.fori_loop(0, _B // 16, chunk, 0)


def _drain_scatter(rows, acc, ssem):
    for j in range(_K):
        pltpu.make_async_copy(rows.at[j], acc.at[pl.ds(0, _B)], ssem).wait()


def _segsum_body(x_hbm, psrc_hbm, pdst_hbm, meta_hbm, out_hbm,
                 srcbuf, dstbuf, dstloc, rows, zbuf, mbuf, acc,
                 gsem, ssem, isem):
    c = lax.axis_index("c")
    s = lax.axis_index("s")
    base = c * _HALF
    iota16 = lax.iota(jnp.int32, 16)
    _zero_acc(zbuf, acc, s, _D)
    plsc.subcore_barrier()

    for r in (2 * s, 2 * s + 1):
        pltpu.sync_copy(meta_hbm.at[r], mbuf)
        mv = mbuf[...]
        n = jnp.sum(jnp.where(iota16 == c, mv, 0))
        base_r = pl.multiple_of(
            jnp.where(c == 0, r * _REG, (r + 1) * _REG - n), 8)
        ng = n // (_B * _K)

        def grp(g, _):
            off = pl.multiple_of(base_r + g * (_B * _K), 8)
            i1 = pltpu.async_copy(psrc_hbm.at[pl.ds(off, _B * _K)], srcbuf,
                                  isem)
            i2 = pltpu.async_copy(pdst_hbm.at[pl.ds(off, _B * _K)], dstbuf,
                                  isem)

            @pl.when(g > 0)
            def _():
                _drain_scatter(rows, acc, ssem)
            i1.wait()
            i2.wait()
            gathers = [
                pltpu.async_copy(x_hbm.at[srcbuf.at[pl.ds(j * _B, _B)]],
                                 rows.at[j], gsem)
                for j in range(_K)
            ]
            for j in range(_K):
                _local_dst(dstbuf, dstloc, j, base)
            for j in range(_K):
                gathers[j].wait()
                pltpu.async_copy(rows.at[j], acc.at[dstloc.at[j]], ssem,
                                 add=True)
            return 0
        lax.fori_loop(0, ng, grp, 0)
        _drain_scatter(rows, acc, ssem)
    plsc.subcore_barrier()
    _writeback(acc, out_hbm, s, base)


def _writeback(acc, out_hbm, s, base):
    nwb = _NCH // _NS + jnp.where(s < (_NCH % _NS), 1, 0)

    def wb(k, _):
        row = (s + _NS * k) * _CW
        pltpu.sync_copy(acc.at[pl.ds(row, _CW)],
                        out_hbm.at[pl.ds(base + row, _CW)])
        return 0
    lax.fori_loop(0, nwb, wb, 0)


_segsum = pl.kernel(
    _segsum_body,
    out_type=jax.ShapeDtypeStruct((_N, _D), jnp.float32),
    mesh=_sc_mesh,
    scratch_types=[
        pltpu.VMEM((_B * _K,), jnp.int32),
        pltpu.VMEM((_B * _K,), jnp.int32),
        pltpu.VMEM((_K, _B), jnp.int32),
        pltpu.VMEM((_K, _B, _D), jnp.float32),
        pltpu.VMEM((_ZROWS, _D), jnp.float32),
        pltpu.VMEM((16,), jnp.int32),
        pltpu.VMEM_SHARED((_ACC_ROWS, _D), jnp.float32),
        pltpu.SemaphoreType.DMA,
        pltpu.SemaphoreType.DMA,
        pltpu.SemaphoreType.DMA,
    ],
    compiler_params=_sc_params_nl,
)


def _part_body(src_hbm, dst_hbm, psrc_hbm, pdst_hbm, meta_hbm,
               sbuf, dbuf, b0s, b0d, b1s, b1d, mbuf):
    c = lax.axis_index("c")
    s = lax.axis_index("s")
    w = s * _NC + c
    wbeg = w * _REG
    wend = (w + 1) * _REG
    iota16 = lax.iota(jnp.int32, 16)

    def chunk(g, carry):
        pltpu.sync_copy(src_hbm.at[pl.ds(w * _EPW + g * _CCH, _CCH)], sbuf)
        pltpu.sync_copy(dst_hbm.at[pl.ds(w * _EPW + g * _CCH, _CCH)], dbuf)

        def inner(m, carry):
            c0, c1, o0, o1 = carry
            s16 = sbuf[pl.ds(16 * m, 16)]
            d16 = dbuf[pl.ds(16 * m, 16)]
            m0 = d16 < _HALF
            m1 = jnp.logical_not(m0)
            inc0 = jnp.where(m0, 1, 0)
            pos0 = c0 + plsc.cumsum(inc0) - 1
            pos1 = c1 + plsc.cumsum(1 - inc0) - 1
            plsc.store_scatter(b0s, [pos0], s16, mask=m0)
            plsc.store_scatter(b0d, [pos0], d16, mask=m0)
            plsc.store_scatter(b1s, [pos1], s16, mask=m1)
            plsc.store_scatter(b1d, [pos1], d16, mask=m1)
            n0 = jnp.sum(inc0)
            c0 = c0 + n0
            c1 = c1 + (16 - n0)
            f0 = c0 >= _F

            @pl.when(f0)
            def _():
                off = pl.multiple_of(wbeg + o0, 8)
                pltpu.sync_copy(b0s.at[pl.ds(0, _F)],
                                psrc_hbm.at[pl.ds(off, _F)])
                pltpu.sync_copy(b0d.at[pl.ds(0, _F)],
                                pdst_hbm.at[pl.ds(off, _F)])
                b0s[pl.ds(0, 16)] = b0s[pl.ds(_F, 16)]
                b0d[pl.ds(0, 16)] = b0d[pl.ds(_F, 16)]
            c0 = jnp.where(f0, c0 - _F, c0)
            o0 = jnp.where(f0, o0 + _F, o0)
            f1 = c1 >= _F

            @pl.when(f1)
            def _():
                off = pl.multiple_of(wend - o1 - _F, 8)
                pltpu.sync_copy(b1s.at[pl.ds(0, _F)],
                                psrc_hbm.at[pl.ds(off, _F)])
                pltpu.sync_copy(b1d.at[pl.ds(0, _F)],
                                pdst_hbm.at[pl.ds(off, _F)])
                b1s[pl.ds(0, 16)] = b1s[pl.ds(_F, 16)]
                b1d[pl.ds(0, 16)] = b1d[pl.ds(_F, 16)]
            c1 = jnp.where(f1, c1 - _F, c1)
            o1 = jnp.where(f1, o1 + _F, o1)
            return (c0, c1, o0, o1)
        return lax.fori_loop(0, _CCH // 16, inner, carry)

    z = jnp.int32(0)
    c0, c1, o0, o1 = lax.fori_loop(0, _EPW // _CCH, chunk, (z, z, z, z))

    def fill(buf, cnt, trashval):
        def f(k, _):
            pos = 16 * k
            cur = buf[pl.ds(pos, 16)]
            sel = (pos + iota16) >= cnt
            buf[pl.ds(pos, 16)] = jnp.where(sel, trashval, cur)
            return 0
        lax.fori_loop(0, _BCAP // 16, f, 0)
    fill(b0s, c0, 0)
    fill(b0d, c0, _N)
    fill(b1s, c1, 0)
    fill(b1d, c1, _N)
    off0 = pl.multiple_of(wbeg + o0, 8)
    pltpu.sync_copy(b0s.at[pl.ds(0, _FF)], psrc_hbm.at[pl.ds(off0, _FF)])
    pltpu.sync_copy(b0d.at[pl.ds(0, _FF)], pdst_hbm.at[pl.ds(off0, _FF)])
    off1 = pl.multiple_of(wend - o1 - _FF, 8)
    pltpu.sync_copy(b1s.at[pl.ds(0, _FF)], psrc_hbm.at[pl.ds(off1, _FF)])
    pltpu.sync_copy(b1d.at[pl.ds(0, _FF)], pdst_hbm.at[pl.ds(off1, _FF)])
    n0 = o0 + _FF
    n1 = o1 + _FF
    mbuf[pl.ds(0, 16)] = jnp.where(iota16 == 0, n0,
                                   jnp.where(iota16 == 1, n1, 0))
    pltpu.sync_copy(mbuf, meta_hbm.at[w])


_part = pl.kernel(
    _part_body,
    out_type=(jax.ShapeDtypeStruct((_EP,), jnp.int32),
              jax.ShapeDtypeStruct((_EP,), jnp.int32),
              jax.ShapeDtypeStruct((_NW, 16), jnp.int32)),
    mesh=_sc_mesh,
    scratch_types=[
        pltpu.VMEM((_CCH,), jnp.int32),
        pltpu.VMEM((_CCH,), jnp.int32),
        pltpu.VMEM((_BCAP,), jnp.int32),
        pltpu.VMEM((_BCAP,), jnp.int32),
        pltpu.VMEM((_BCAP,), jnp.int32),
        pltpu.VMEM((_BCAP,), jnp.int32),
        pltpu.VMEM((16,), jnp.int32),
    ],
    compiler_params=_sc_params_nl,
)


_HPAD = 102400


def _count_body(dst_hbm, out_hbm, dstbuf, hist):
    c = lax.axis_index("c")
    s = lax.axis_index("s")
    w = s * _NC + c
    ones = jnp.ones((16,), jnp.float32)

    def zero(i, _):
        hist[pl.ds(16 * i, 16)] = jnp.zeros((16,), jnp.float32)
        return 0
    lax.fori_loop(0, _HPAD // 16, zero, 0)

    def chunk(g, _):
        pltpu.sync_copy(dst_hbm.at[pl.ds(w * _EPW + g * _CCH, _CCH)], dstbuf)

        def upd(m, _):
            d = dstbuf[pl.ds(16 * m, 16)]
            plsc.addupdate_scatter(hist, [d], ones)
            return 0
        lax.fori_loop(0, _CCH // 16, upd, 0)
        return 0
    lax.fori_loop(0, _EPW // _CCH, chunk, 0)
    pltpu.sync_copy(hist, out_hbm.at[w])


_count = pl.kernel(
    _count_body,
    out_type=jax.ShapeDtypeStruct((_NW, _HPAD), jnp.float32),
    mesh=_sc_mesh,
    scratch_types=[
        pltpu.VMEM((_CCH,), jnp.int32),
        pltpu.VMEM((_HPAD,), jnp.float32),
    ],
    compiler_params=_sc_params_nl,
)


def _hsum_body(p_ref, o_ref):
    o_ref[...] = jnp.sum(p_ref[...], axis=0)


_hsum = pl.pallas_call(
    _hsum_body,
    grid=(_HPAD // 2048,),
    in_specs=[pl.BlockSpec((_NW, 2048), lambda i: (0, i))],
    out_specs=pl.BlockSpec((2048,), lambda i: (i,)),
    out_shape=jax.ShapeDtypeStruct((_HPAD,), jnp.float32),
)


def _matT(a, w):
    return lax.dot_general(a, w, (((1,), (1,)), ((), ())),
                           preferred_element_type=jnp.float32)


def _dense1_body(s_ref, cnt_ref, x_ref, wl_ref, bl_ref, wr_ref, o_ref):
    r = 1.0 / jnp.maximum(cnt_ref[...], 1.0)
    mean = s_ref[...] * r
    o_ref[...] = jnp.maximum(
        _matT(mean, wl_ref[...]) + _matT(x_ref[...], wr_ref[...]) + bl_ref[...], 0.0)


def _dense2_body(s_ref, cnt_ref, h_ref, wl_ref, bl_ref, wr_ref,
                 wlin_ref, blin_ref, nh_ref, g_ref):
    i = pl.program_id(0)
    r = 1.0 / jnp.maximum(cnt_ref[...], 1.0)
    mean = s_ref[...] * r
    h1 = jnp.maximum(
        _matT(mean, wl_ref[...]) + _matT(h_ref[...], wr_ref[...]) + bl_ref[...], 0.0)
    nh = jnp.maximum(_matT(h1, wlin_ref[...]) + blin_ref[...], 0.0)
    nh_ref[...] = nh

    @pl.when(i == 0)
    def _():
        g_ref[...] = jnp.zeros_like(g_ref)
    g_ref[...] += jnp.sum(nh, axis=0, keepdims=True)

    @pl.when(i == pl.num_programs(0) - 1)
    def _():
        g_ref[...] *= (1.0 / _N)


_ROWS_TC = 2000
_GRID = _N // _ROWS_TC

_row_spec = pl.BlockSpec((_ROWS_TC, _D), lambda i: (i, 0))
_cnt_spec = pl.BlockSpec((_ROWS_TC, 1), lambda i: (i, 0))
_w_spec = pl.BlockSpec((_D, _D), lambda i: (0, 0))
_b_spec = pl.BlockSpec((1, _D), lambda i: (0, 0))

_dense1 = pl.pallas_call(
    _dense1_body,
    grid=(_GRID,),
    in_specs=[_row_spec, _cnt_spec, _row_spec, _w_spec, _b_spec, _w_spec],
    out_specs=_row_spec,
    out_shape=jax.ShapeDtypeStruct((_N, _D), jnp.float32),
)

_dense2 = pl.pallas_call(
    _dense2_body,
    grid=(_GRID,),
    in_specs=[_row_spec, _cnt_spec, _row_spec, _w_spec, _b_spec, _w_spec,
              _w_spec, _b_spec],
    out_specs=[_row_spec, _b_spec],
    out_shape=[jax.ShapeDtypeStruct((_N, _D), jnp.float32),
               jax.ShapeDtypeStruct((1, _D), jnp.float32)],
)


@jax.jit
def kernel(x, edge_index, batch, Wl0, bl0, Wr0, Wl1, bl1, Wr1, Wlin, blin):
    src = edge_index[0]
    dst = edge_index[1]
    psrc, pdst, meta = _part(src, dst)
    cnt = _hsum(_count(dst))[:_N].reshape(_N, 1)
    s0 = _segsum(x, psrc, pdst, meta)
    h0 = _dense1(s0, cnt, x, Wl0, bl0.reshape(1, _D), Wr0)
    s1 = _segsum(h0, psrc, pdst, meta)
    node_h, g = _dense2(s1, cnt, h0, Wl1, bl1.reshape(1, _D), Wr1,
                        Wlin, blin.reshape(1, _D))
    return node_h, g


# speedup vs baseline: 1.1712x; 1.1712x over previous
import functools

import jax
import jax.numpy as jnp
from jax import lax
from jax.experimental import pallas as pl
from jax.experimental.pallas import tpu as pltpu
from jax.experimental.pallas import tpu_sc as plsc

_N = 100000
_E = 1600000
_D = 32
_NC = 2
_NS = 16
_HALF = _N // _NC
_ACC_ROWS = 50176
_ZROWS = 56
_B = 80
_K = 5
_EPT = _E // _NS
_NGRP = _EPT // (_B * _K)
_NW = _NC * _NS
_CCH = 2000
_F = 800
_FF = 1200
_BCAP = 1216
_REG = 52800
_EPW = _E // (_NC * _NS)
_EP = _NC * _NS * _REG
_CW = 400
_NCH = _HALF // _CW
_CNT_W = 16

_sc_mesh = plsc.VectorSubcoreMesh(core_axis_name="c", subcore_axis_name="s")
_sc_params = pltpu.CompilerParams(use_tc_tiling_on_sc=False)
_sc_params_nl = pltpu.CompilerParams(use_tc_tiling_on_sc=False,
                                     needs_layout_passes=False)


def _zero_acc(zbuf, acc, s, width):
    def zrow(k, _):
        for j in range(width // 16):
            zbuf[k, pl.ds(16 * j, 16)] = jnp.zeros((16,), jnp.float32)
        return 0
    lax.fori_loop(0, _ZROWS, zrow, 0)
    rows_pt = _ACC_ROWS // _NS

    def zcpy(k, _):
        pltpu.sync_copy(zbuf, acc.at[pl.ds(s * rows_pt + k * _ZROWS, _ZROWS)])
        return 0
    lax.fori_loop(0, rows_pt // _ZROWS, zcpy, 0)


def _local_dst(dstbuf, dstloc, j, base):
    def chunk(m, _):
        d = dstbuf[pl.ds(j * _B + 16 * m, 16)]
        local = d - base
        ok = (local >= 0) & (local < _HALF)
        dstloc[j, pl.ds(16 * m, 16)] = jnp.where(ok, local, _HALF)
        return 0
    lax.fori_loop(0, _B // 16, chunk, 0)


def _drain_scatter(rows, acc, ssem):
    for j in range(_K):
        pltpu.make_async_copy(rows.at[j], acc.at[pl.ds(0, _B)], ssem).wait()


def _segsum_body(x_hbm, psrc_hbm, pdst_hbm, meta_hbm, out_hbm,
                 srcbuf, dstbuf, dstloc, rows, zbuf, mbuf, acc,
                 gsem, ssem):
    c = lax.axis_index("c")
    s = lax.axis_index("s")
    base = c * _HALF
    iota16 = lax.iota(jnp.int32, 16)
    _zero_acc(zbuf, acc, s, _D)
    plsc.subcore_barrier()

    for r in (2 * s, 2 * s + 1):
        pltpu.sync_copy(meta_hbm.at[r], mbuf)
        mv = mbuf[...]
        n = jnp.sum(jnp.where(iota16 == c, mv, 0))
        base_r = pl.multiple_of(
            jnp.where(c == 0, r * _REG, (r + 1) * _REG - n), 8)
        ng = n // (_B * _K)

        def grp(g, _):
            off = pl.multiple_of(base_r + g * (_B * _K), 8)
            pltpu.sync_copy(psrc_hbm.at[pl.ds(off, _B * _K)], srcbuf)
            pltpu.sync_copy(pdst_hbm.at[pl.ds(off, _B * _K)], dstbuf)

            @pl.when(g > 0)
            def _():
                _drain_scatter(rows, acc, ssem)
            gathers = [
                pltpu.async_copy(x_hbm.at[srcbuf.at[pl.ds(j * _B, _B)]],
                                 rows.at[j], gsem)
                for j in range(_K)
            ]
            for j in range(_K):
                _local_dst(dstbuf, dstloc, j, base)
            for j in range(_K):
                gathers[j].wait()
                pltpu.async_copy(rows.at[j], acc.at[dstloc.at[j]], ssem,
                                 add=True)
            return 0
        lax.fori_loop(0, ng, grp, 0)
        _drain_scatter(rows, acc, ssem)
    plsc.subcore_barrier()
    _writeback(acc, out_hbm, s, base)


def _writeback(acc, out_hbm, s, base):
    nwb = _NCH // _NS + jnp.where(s < (_NCH % _NS), 1, 0)

    def wb(k, _):
        row = (s + _NS * k) * _CW
        pltpu.sync_copy(acc.at[pl.ds(row, _CW)],
                        out_hbm.at[pl.ds(base + row, _CW)])
        return 0
    lax.fori_loop(0, nwb, wb, 0)


_segsum = pl.kernel(
    _segsum_body,
    out_type=jax.ShapeDtypeStruct((_N, _D), jnp.float32),
    mesh=_sc_mesh,
    scratch_types=[
        pltpu.VMEM((_B * _K,), jnp.int32),
        pltpu.VMEM((_B * _K,), jnp.int32),
        pltpu.VMEM((_K, _B), jnp.int32),
        pltpu.VMEM((_K, _B, _D), jnp.float32),
        pltpu.VMEM((_ZROWS, _D), jnp.float32),
        pltpu.VMEM((16,), jnp.int32),
        pltpu.VMEM_SHARED((_ACC_ROWS, _D), jnp.float32),
        pltpu.SemaphoreType.DMA,
        pltpu.SemaphoreType.DMA,
    ],
    compiler_params=_sc_params_nl,
)


def _part_body(src_hbm, dst_hbm, psrc_hbm, pdst_hbm, meta_hbm,
               sbuf, dbuf, b0s, b0d, b1s, b1d, mbuf):
    c = lax.axis_index("c")
    s = lax.axis_index("s")
    w = s * _NC + c
    wbeg = w * _REG
    wend = (w + 1) * _REG
    iota16 = lax.iota(jnp.int32, 16)

    def chunk(g, carry):
        pltpu.sync_copy(src_hbm.at[pl.ds(w * _EPW + g * _CCH, _CCH)], sbuf)
        pltpu.sync_copy(dst_hbm.at[pl.ds(w * _EPW + g * _CCH, _CCH)], dbuf)

        def inner(m, carry):
            c0, c1, o0, o1 = carry
            s16 = sbuf[pl.ds(16 * m, 16)]
            d16 = dbuf[pl.ds(16 * m, 16)]
            m0 = d16 < _HALF
            m1 = jnp.logical_not(m0)
            inc0 = jnp.where(m0, 1, 0)
            pos0 = c0 + plsc.cumsum(inc0) - 1
            pos1 = c1 + plsc.cumsum(1 - inc0) - 1
            plsc.store_scatter(b0s, [pos0], s16, mask=m0)
            plsc.store_scatter(b0d, [pos0], d16, mask=m0)
            plsc.store_scatter(b1s, [pos1], s16, mask=m1)
            plsc.store_scatter(b1d, [pos1], d16, mask=m1)
            n0 = jnp.sum(inc0)
            c0 = c0 + n0
            c1 = c1 + (16 - n0)
            f0 = c0 >= _F

            @pl.when(f0)
            def _():
                off = pl.multiple_of(wbeg + o0, 8)
                pltpu.sync_copy(b0s.at[pl.ds(0, _F)],
                                psrc_hbm.at[pl.ds(off, _F)])
                pltpu.sync_copy(b0d.at[pl.ds(0, _F)],
                                pdst_hbm.at[pl.ds(off, _F)])
                b0s[pl.ds(0, 16)] = b0s[pl.ds(_F, 16)]
                b0d[pl.ds(0, 16)] = b0d[pl.ds(_F, 16)]
            c0 = jnp.where(f0, c0 - _F, c0)
            o0 = jnp.where(f0, o0 + _F, o0)
            f1 = c1 >= _F

            @pl.when(f1)
            def _():
                off = pl.multiple_of(wend - o1 - _F, 8)
                pltpu.sync_copy(b1s.at[pl.ds(0, _F)],
                                psrc_hbm.at[pl.ds(off, _F)])
                pltpu.sync_copy(b1d.at[pl.ds(0, _F)],
                                pdst_hbm.at[pl.ds(off, _F)])
                b1s[pl.ds(0, 16)] = b1s[pl.ds(_F, 16)]
                b1d[pl.ds(0, 16)] = b1d[pl.ds(_F, 16)]
            c1 = jnp.where(f1, c1 - _F, c1)
            o1 = jnp.where(f1, o1 + _F, o1)
            return (c0, c1, o0, o1)
        return lax.fori_loop(0, _CCH // 16, inner, carry)

    z = jnp.int32(0)
    c0, c1, o0, o1 = lax.fori_loop(0, _EPW // _CCH, chunk, (z, z, z, z))

    def fill(buf, cnt, trashval):
        def f(k, _):
            pos = 16 * k
            cur = buf[pl.ds(pos, 16)]
            sel = (pos + iota16) >= cnt
            buf[pl.ds(pos, 16)] = jnp.where(sel, trashval, cur)
            return 0
        lax.fori_loop(0, _BCAP // 16, f, 0)
    fill(b0s, c0, 0)
    fill(b0d, c0, _N)
    fill(b1s, c1, 0)
    fill(b1d, c1, _N)
    off0 = pl.multiple_of(wbeg + o0, 8)
    pltpu.sync_copy(b0s.at[pl.ds(0, _FF)], psrc_hbm.at[pl.ds(off0, _FF)])
    pltpu.sync_copy(b0d.at[pl.ds(0, _FF)], pdst_hbm.at[pl.ds(off0, _FF)])
    off1 = pl.multiple_of(wend - o1 - _FF, 8)
    pltpu.sync_copy(b1s.at[pl.ds(0, _FF)], psrc_hbm.at[pl.ds(off1, _FF)])
    pltpu.sync_copy(b1d.at[pl.ds(0, _FF)], pdst_hbm.at[pl.ds(off1, _FF)])
    n0 = o0 + _FF
    n1 = o1 + _FF
    mbuf[pl.ds(0, 16)] = jnp.where(iota16 == 0, n0,
                                   jnp.where(iota16 == 1, n1, 0))
    pltpu.sync_copy(mbuf, meta_hbm.at[w])


_part = pl.kernel(
    _part_body,
    out_type=(jax.ShapeDtypeStruct((_EP,), jnp.int32),
              jax.ShapeDtypeStruct((_EP,), jnp.int32),
              jax.ShapeDtypeStruct((_NW, 16), jnp.int32)),
    mesh=_sc_mesh,
    scratch_types=[
        pltpu.VMEM((_CCH,), jnp.int32),
        pltpu.VMEM((_CCH,), jnp.int32),
        pltpu.VMEM((_BCAP,), jnp.int32),
        pltpu.VMEM((_BCAP,), jnp.int32),
        pltpu.VMEM((_BCAP,), jnp.int32),
        pltpu.VMEM((_BCAP,), jnp.int32),
        pltpu.VMEM((16,), jnp.int32),
    ],
    compiler_params=_sc_params_nl,
)


_HPAD = 102400


def _count_body(dst_hbm, out_hbm, dstbuf, hist):
    c = lax.axis_index("c")
    s = lax.axis_index("s")
    w = s * _NC + c
    ones = jnp.ones((16,), jnp.float32)

    def zero(i, _):
        hist[pl.ds(16 * i, 16)] = jnp.zeros((16,), jnp.float32)
        return 0
    lax.fori_loop(0, _HPAD // 16, zero, 0)

    def chunk(g, _):
        pltpu.sync_copy(dst_hbm.at[pl.ds(w * _EPW + g * _CCH, _CCH)], dstbuf)

        def upd(m, _):
            d = dstbuf[pl.ds(16 * m, 16)]
            plsc.addupdate_scatter(hist, [d], ones)
            return 0
        lax.fori_loop(0, _CCH // 16, upd, 0)
        return 0
    lax.fori_loop(0, _EPW // _CCH, chunk, 0)
    pltpu.sync_copy(hist, out_hbm.at[w])


_count = pl.kernel(
    _count_body,
    out_type=jax.ShapeDtypeStruct((_NW, _HPAD), jnp.float32),
    mesh=_sc_mesh,
    scratch_types=[
        pltpu.VMEM((_CCH,), jnp.int32),
        pltpu.VMEM((_HPAD,), jnp.float32),
    ],
    compiler_params=_sc_params_nl,
)


def _hsum_body(p_ref, o_ref):
    o_ref[...] = jnp.sum(p_ref[...], axis=0)


_hsum = pl.pallas_call(
    _hsum_body,
    grid=(_HPAD // 2048,),
    in_specs=[pl.BlockSpec((_NW, 2048), lambda i: (0, i))],
    out_specs=pl.BlockSpec((2048,), lambda i: (i,)),
    out_shape=jax.ShapeDtypeStruct((_HPAD,), jnp.float32),
)


def _matT(a, w):
    return lax.dot_general(a, w, (((1,), (1,)), ((), ())),
                           preferred_element_type=jnp.float32)


def _dense1_body(s_ref, cnt_ref, x_ref, wl_ref, bl_ref, wr_ref, o_ref):
    r = 1.0 / jnp.maximum(cnt_ref[...], 1.0)
    mean = s_ref[...] * r
    o_ref[...] = jnp.maximum(
        _matT(mean, wl_ref[...]) + _matT(x_ref[...], wr_ref[...]) + bl_ref[...], 0.0)


def _dense2_body(s_ref, cnt_ref, h_ref, wl_ref, bl_ref, wr_ref,
                 wlin_ref, blin_ref, nh_ref, g_ref):
    i = pl.program_id(0)
    r = 1.0 / jnp.maximum(cnt_ref[...], 1.0)
    mean = s_ref[...] * r
    h1 = jnp.maximum(
        _matT(mean, wl_ref[...]) + _matT(h_ref[...], wr_ref[...]) + bl_ref[...], 0.0)
    nh = jnp.maximum(_matT(h1, wlin_ref[...]) + blin_ref[...], 0.0)
    nh_ref[...] = nh

    @pl.when(i == 0)
    def _():
        g_ref[...] = jnp.zeros_like(g_ref)
    g_ref[...] += jnp.sum(nh, axis=0, keepdims=True)

    @pl.when(i == pl.num_programs(0) - 1)
    def _():
        g_ref[...] *= (1.0 / _N)


_ROWS_TC = 2000
_GRID = _N // _ROWS_TC

_row_spec = pl.BlockSpec((_ROWS_TC, _D), lambda i: (i, 0))
_cnt_spec = pl.BlockSpec((_ROWS_TC, 1), lambda i: (i, 0))
_w_spec = pl.BlockSpec((_D, _D), lambda i: (0, 0))
_b_spec = pl.BlockSpec((1, _D), lambda i: (0, 0))

_dense1 = pl.pallas_call(
    _dense1_body,
    grid=(_GRID,),
    in_specs=[_row_spec, _cnt_spec, _row_spec, _w_spec, _b_spec, _w_spec],
    out_specs=_row_spec,
    out_shape=jax.ShapeDtypeStruct((_N, _D), jnp.float32),
)

_dense2 = pl.pallas_call(
    _dense2_body,
    grid=(_GRID,),
    in_specs=[_row_spec, _cnt_spec, _row_spec, _w_spec, _b_spec, _w_spec,
              _w_spec, _b_spec],
    out_specs=[_row_spec, _b_spec],
    out_shape=[jax.ShapeDtypeStruct((_N, _D), jnp.float32),
               jax.ShapeDtypeStruct((1, _D), jnp.float32)],
)


@jax.jit
def kernel(x, edge_index, batch, Wl0, bl0, Wr0, Wl1, bl1, Wr1, Wlin, blin):
    src = edge_index[0]
    dst = edge_index[1]
    psrc, pdst, meta = _part(src, dst)
    cnt = _hsum(_count(dst))[:_N].reshape(_N, 1)
    s0 = _segsum(x, psrc, pdst, meta)
    h0 = _dense1(s0, cnt, x, Wl0, bl0.reshape(1, _D), Wr0)
    s1 = _segsum(h0, psrc, pdst, meta)
    node_h, g = _dense2(s1, cnt, h0, Wl1, bl1.reshape(1, _D), Wr1,
                        Wlin, blin.reshape(1, _D))
    return node_h, g


# speedup vs baseline: 1.2287x; 1.0491x over previous
import functools

import jax
import jax.numpy as jnp
from jax import lax
from jax.experimental import pallas as pl
from jax.experimental.pallas import tpu as pltpu
from jax.experimental.pallas import tpu_sc as plsc

_N = 100000
_E = 1600000
_D = 32
_NC = 2
_NS = 16
_HALF = _N // _NC
_ACC_ROWS = 50176
_ZROWS = 56
_B = 80
_K = 5
_EPT = _E // _NS
_NGRP = _EPT // (_B * _K)
_NW = _NC * _NS
_CCH = 2000
_HPAD = 102400
_F = 800
_FF = 1200
_BCAP = 1216
_REG = 52800
_EPW = _E // (_NC * _NS)
_EP = _NC * _NS * _REG
_CW = 400
_NCH = _HALF // _CW
_CNT_W = 16

_sc_mesh = plsc.VectorSubcoreMesh(core_axis_name="c", subcore_axis_name="s")
_sc_params = pltpu.CompilerParams(use_tc_tiling_on_sc=False)
_sc_params_nl = pltpu.CompilerParams(use_tc_tiling_on_sc=False,
                                     needs_layout_passes=False)


def _zero_acc(zbuf, acc, s, width):
    def zrow(k, _):
        for j in range(width // 16):
            zbuf[k, pl.ds(16 * j, 16)] = jnp.zeros((16,), jnp.float32)
        return 0
    lax.fori_loop(0, _ZROWS, zrow, 0)
    rows_pt = _ACC_ROWS // _NS

    def zcpy(k, _):
        pltpu.sync_copy(zbuf, acc.at[pl.ds(s * rows_pt + k * _ZROWS, _ZROWS)])
        return 0
    lax.fori_loop(0, rows_pt // _ZROWS, zcpy, 0)


def _drain_scatter(rows, acc, ssem):
    for j in range(_K):
        pltpu.make_async_copy(rows.at[j], acc.at[pl.ds(0, _B)], ssem).wait()


def _segsum_body(x_hbm, psrc_hbm, pdst_hbm, meta_hbm, out_hbm,
                 srcbuf, dstloc, rows, zbuf, mbuf, acc,
                 gsem, ssem, isem):
    c = lax.axis_index("c")
    s = lax.axis_index("s")
    base = c * _HALF
    iota16 = lax.iota(jnp.int32, 16)
    _zero_acc(zbuf, acc, s, _D)
    plsc.subcore_barrier()

    for r in (2 * s, 2 * s + 1):
        pltpu.sync_copy(meta_hbm.at[r], mbuf)
        mv = mbuf[...]
        n = jnp.sum(jnp.where(iota16 == c, mv, 0))
        base_r = pl.multiple_of(
            jnp.where(c == 0, r * _REG, (r + 1) * _REG - n), 8)
        ng = n // (_B * _K)

        def grp(g, _):
            off = pl.multiple_of(base_r + g * (_B * _K), 8)
            pltpu.sync_copy(psrc_hbm.at[pl.ds(off, _B * _K)], srcbuf)
            idxs = [
                pltpu.async_copy(pdst_hbm.at[pl.ds(off + j * _B, _B)],
                                 dstloc.at[j], isem)
                for j in range(_K)
            ]

            @pl.when(g > 0)
            def _():
                _drain_scatter(rows, acc, ssem)
            gathers = [
                pltpu.async_copy(x_hbm.at[srcbuf.at[pl.ds(j * _B, _B)]],
                                 rows.at[j], gsem)
                for j in range(_K)
            ]
            for j in range(_K):
                idxs[j].wait()
            for j in range(_K):
                gathers[j].wait()
                pltpu.async_copy(rows.at[j], acc.at[dstloc.at[j]], ssem,
                                 add=True)
            return 0
        lax.fori_loop(0, ng, grp, 0)
        _drain_scatter(rows, acc, ssem)
    plsc.subcore_barrier()
    _writeback(acc, out_hbm, s, base)


def _writeback(acc, out_hbm, s, base):
    nwb = _NCH // _NS + jnp.where(s < (_NCH % _NS), 1, 0)

    def wb(k, _):
        row = (s + _NS * k) * _CW
        pltpu.sync_copy(acc.at[pl.ds(row, _CW)],
                        out_hbm.at[pl.ds(base + row, _CW)])
        return 0
    lax.fori_loop(0, nwb, wb, 0)


_segsum = pl.kernel(
    _segsum_body,
    out_type=jax.ShapeDtypeStruct((_N, _D), jnp.float32),
    mesh=_sc_mesh,
    scratch_types=[
        pltpu.VMEM((_B * _K,), jnp.int32),
        pltpu.VMEM((_K, _B), jnp.int32),
        pltpu.VMEM((_K, _B, _D), jnp.float32),
        pltpu.VMEM((_ZROWS, _D), jnp.float32),
        pltpu.VMEM((16,), jnp.int32),
        pltpu.VMEM_SHARED((_ACC_ROWS, _D), jnp.float32),
        pltpu.SemaphoreType.DMA,
        pltpu.SemaphoreType.DMA,
        pltpu.SemaphoreType.DMA,
    ],
    compiler_params=_sc_params_nl,
)


def _part_body(src_hbm, dst_hbm, psrc_hbm, pdst_hbm, meta_hbm, hist_hbm,
               sbuf, dbuf, b0s, b0d, b1s, b1d, mbuf, hist):
    c = lax.axis_index("c")
    s = lax.axis_index("s")
    w = s * _NC + c
    wbeg = w * _REG
    wend = (w + 1) * _REG
    iota16 = lax.iota(jnp.int32, 16)
    ones16 = jnp.ones((16,), jnp.float32)

    def hzero(i, _):
        hist[pl.ds(16 * i, 16)] = jnp.zeros((16,), jnp.float32)
        return 0
    lax.fori_loop(0, _HPAD // 16, hzero, 0)

    def chunk(g, carry):
        pltpu.sync_copy(src_hbm.at[pl.ds(w * _EPW + g * _CCH, _CCH)], sbuf)
        pltpu.sync_copy(dst_hbm.at[pl.ds(w * _EPW + g * _CCH, _CCH)], dbuf)

        def inner(m, carry):
            c0, c1, o0, o1 = carry
            s16 = sbuf[pl.ds(16 * m, 16)]
            d16 = dbuf[pl.ds(16 * m, 16)]
            m0 = d16 < _HALF
            m1 = jnp.logical_not(m0)
            plsc.addupdate_scatter(hist, [d16], ones16)
            inc0 = jnp.where(m0, 1, 0)
            pos0 = c0 + plsc.cumsum(inc0) - 1
            pos1 = c1 + plsc.cumsum(1 - inc0) - 1
            plsc.store_scatter(b0s, [pos0], s16, mask=m0)
            plsc.store_scatter(b0d, [pos0], d16, mask=m0)
            plsc.store_scatter(b1s, [pos1], s16, mask=m1)
            plsc.store_scatter(b1d, [pos1], d16 - _HALF, mask=m1)
            n0 = jnp.sum(inc0)
            c0 = c0 + n0
            c1 = c1 + (16 - n0)
            f0 = c0 >= _F

            @pl.when(f0)
            def _():
                off = pl.multiple_of(wbeg + o0, 8)
                pltpu.sync_copy(b0s.at[pl.ds(0, _F)],
                                psrc_hbm.at[pl.ds(off, _F)])
                pltpu.sync_copy(b0d.at[pl.ds(0, _F)],
                                pdst_hbm.at[pl.ds(off, _F)])
                b0s[pl.ds(0, 16)] = b0s[pl.ds(_F, 16)]
                b0d[pl.ds(0, 16)] = b0d[pl.ds(_F, 16)]
            c0 = jnp.where(f0, c0 - _F, c0)
            o0 = jnp.where(f0, o0 + _F, o0)
            f1 = c1 >= _F

            @pl.when(f1)
            def _():
                off = pl.multiple_of(wend - o1 - _F, 8)
                pltpu.sync_copy(b1s.at[pl.ds(0, _F)],
                                psrc_hbm.at[pl.ds(off, _F)])
                pltpu.sync_copy(b1d.at[pl.ds(0, _F)],
                                pdst_hbm.at[pl.ds(off, _F)])
                b1s[pl.ds(0, 16)] = b1s[pl.ds(_F, 16)]
                b1d[pl.ds(0, 16)] = b1d[pl.ds(_F, 16)]
            c1 = jnp.where(f1, c1 - _F, c1)
            o1 = jnp.where(f1, o1 + _F, o1)
            return (c0, c1, o0, o1)
        return lax.fori_loop(0, _CCH // 16, inner, carry)

    z = jnp.int32(0)
    c0, c1, o0, o1 = lax.fori_loop(0, _EPW // _CCH, chunk, (z, z, z, z))

    def fill(buf, cnt, trashval):
        def f(k, _):
            pos = 16 * k
            cur = buf[pl.ds(pos, 16)]
            sel = (pos + iota16) >= cnt
            buf[pl.ds(pos, 16)] = jnp.where(sel, trashval, cur)
            return 0
        lax.fori_loop(0, _BCAP // 16, f, 0)
    fill(b0s, c0, 0)
    fill(b0d, c0, _HALF)
    fill(b1s, c1, 0)
    fill(b1d, c1, _HALF)
    off0 = pl.multiple_of(wbeg + o0, 8)
    pltpu.sync_copy(b0s.at[pl.ds(0, _FF)], psrc_hbm.at[pl.ds(off0, _FF)])
    pltpu.sync_copy(b0d.at[pl.ds(0, _FF)], pdst_hbm.at[pl.ds(off0, _FF)])
    off1 = pl.multiple_of(wend - o1 - _FF, 8)
    pltpu.sync_copy(b1s.at[pl.ds(0, _FF)], psrc_hbm.at[pl.ds(off1, _FF)])
    pltpu.sync_copy(b1d.at[pl.ds(0, _FF)], pdst_hbm.at[pl.ds(off1, _FF)])
    n0 = o0 + _FF
    n1 = o1 + _FF
    mbuf[pl.ds(0, 16)] = jnp.where(iota16 == 0, n0,
                                   jnp.where(iota16 == 1, n1, 0))
    pltpu.sync_copy(mbuf, meta_hbm.at[w])
    pltpu.sync_copy(hist, hist_hbm.at[w])


_part = pl.kernel(
    _part_body,
    out_type=(jax.ShapeDtypeStruct((_EP,), jnp.int32),
              jax.ShapeDtypeStruct((_EP,), jnp.int32),
              jax.ShapeDtypeStruct((_NW, 16), jnp.int32),
              jax.ShapeDtypeStruct((_NW, _HPAD), jnp.float32)),
    mesh=_sc_mesh,
    scratch_types=[
        pltpu.VMEM((_CCH,), jnp.int32),
        pltpu.VMEM((_CCH,), jnp.int32),
        pltpu.VMEM((_BCAP,), jnp.int32),
        pltpu.VMEM((_BCAP,), jnp.int32),
        pltpu.VMEM((_BCAP,), jnp.int32),
        pltpu.VMEM((_BCAP,), jnp.int32),
        pltpu.VMEM((16,), jnp.int32),
        pltpu.VMEM((_HPAD,), jnp.float32),
    ],
    compiler_params=_sc_params_nl,
)


def _hsum_body(p_ref, o_ref):
    o_ref[...] = jnp.sum(p_ref[...], axis=0)


_hsum = pl.pallas_call(
    _hsum_body,
    grid=(_HPAD // 2048,),
    in_specs=[pl.BlockSpec((_NW, 2048), lambda i: (0, i))],
    out_specs=pl.BlockSpec((2048,), lambda i: (i,)),
    out_shape=jax.ShapeDtypeStruct((_HPAD,), jnp.float32),
)


def _matT(a, w):
    return lax.dot_general(a, w, (((1,), (1,)), ((), ())),
                           preferred_element_type=jnp.float32)


def _dense1_body(s_ref, cnt_ref, x_ref, wl_ref, bl_ref, wr_ref, o_ref):
    r = 1.0 / jnp.maximum(cnt_ref[...], 1.0)
    mean = s_ref[...] * r
    o_ref[...] = jnp.maximum(
        _matT(mean, wl_ref[...]) + _matT(x_ref[...], wr_ref[...]) + bl_ref[...], 0.0)


def _dense2_body(s_ref, cnt_ref, h_ref, wl_ref, bl_ref, wr_ref,
                 wlin_ref, blin_ref, nh_ref, g_ref):
    i = pl.program_id(0)
    r = 1.0 / jnp.maximum(cnt_ref[...], 1.0)
    mean = s_ref[...] * r
    h1 = jnp.maximum(
        _matT(mean, wl_ref[...]) + _matT(h_ref[...], wr_ref[...]) + bl_ref[...], 0.0)
    nh = jnp.maximum(_matT(h1, wlin_ref[...]) + blin_ref[...], 0.0)
    nh_ref[...] = nh

    @pl.when(i == 0)
    def _():
        g_ref[...] = jnp.zeros_like(g_ref)
    g_ref[...] += jnp.sum(nh, axis=0, keepdims=True)

    @pl.when(i == pl.num_programs(0) - 1)
    def _():
        g_ref[...] *= (1.0 / _N)


_ROWS_TC = 2000
_GRID = _N // _ROWS_TC

_row_spec = pl.BlockSpec((_ROWS_TC, _D), lambda i: (i, 0))
_cnt_spec = pl.BlockSpec((_ROWS_TC, 1), lambda i: (i, 0))
_w_spec = pl.BlockSpec((_D, _D), lambda i: (0, 0))
_b_spec = pl.BlockSpec((1, _D), lambda i: (0, 0))

_dense1 = pl.pallas_call(
    _dense1_body,
    grid=(_GRID,),
    in_specs=[_row_spec, _cnt_spec, _row_spec, _w_spec, _b_spec, _w_spec],
    out_specs=_row_spec,
    out_shape=jax.ShapeDtypeStruct((_N, _D), jnp.float32),
)

_dense2 = pl.pallas_call(
    _dense2_body,
    grid=(_GRID,),
    in_specs=[_row_spec, _cnt_spec, _row_spec, _w_spec, _b_spec, _w_spec,
              _w_spec, _b_spec],
    out_specs=[_row_spec, _b_spec],
    out_shape=[jax.ShapeDtypeStruct((_N, _D), jnp.float32),
               jax.ShapeDtypeStruct((1, _D), jnp.float32)],
)


@jax.jit
def kernel(x, edge_index, batch, Wl0, bl0, Wr0, Wl1, bl1, Wr1, Wlin, blin):
    src = edge_index[0]
    dst = edge_index[1]
    psrc, pdst, meta, hist = _part(src, dst)
    cnt = _hsum(hist)[:_N].reshape(_N, 1)
    s0 = _segsum(x, psrc, pdst, meta)
    h0 = _dense1(s0, cnt, x, Wl0, bl0.reshape(1, _D), Wr0)
    s1 = _segsum(h0, psrc, pdst, meta)
    node_h, g = _dense2(s1, cnt, h0, Wl1, bl1.reshape(1, _D), Wr1,
                        Wlin, blin.reshape(1, _D))
    return node_h, g
